# baseline (speedup 1.0000x reference)
import functools

import jax
import jax.numpy as jnp
from jax import lax
from jax.experimental import pallas as pl
from jax.experimental.pallas import tpu as pltpu
from jax.experimental.pallas import tpu_sc as plsc

N_ROWS = 100000
N_FACTORS = 64
BATCH = 16384

ROW_BLK = 2000

NUM_WORKERS = 32
B_PER_W = BATCH // NUM_WORKERS
CHUNK = 128
N_CHUNKS = B_PER_W // CHUNK
LANES = 16


def _scores_body(dis_ref, gene_ref, w_ref, b_ref, u_ref, v_ref):
    w = w_ref[...]
    wd = w[:, :N_FACTORS]
    wg = w[:, N_FACTORS:]
    x = dis_ref[...]
    y = gene_ref[...]
    u_ref[...] = jnp.sum(x * wd, axis=1, keepdims=True) + b_ref[0]
    v_ref[...] = jnp.sum(y * wg, axis=1, keepdims=True)


def _compute_scores(disease_table, gene_table, W, b):
    grid = N_ROWS // ROW_BLK
    u, v = pl.pallas_call(
        _scores_body,
        grid=(grid,),
        in_specs=[
            pl.BlockSpec((ROW_BLK, N_FACTORS), lambda i: (i, 0)),
            pl.BlockSpec((ROW_BLK, N_FACTORS), lambda i: (i, 0)),
            pl.BlockSpec((1, 2 * N_FACTORS), lambda i: (0, 0)),
            pl.BlockSpec(memory_space=pltpu.SMEM),
        ],
        out_specs=[
            pl.BlockSpec((ROW_BLK, 1), lambda i: (i, 0)),
            pl.BlockSpec((ROW_BLK, 1), lambda i: (i, 0)),
        ],
        out_shape=[
            jax.ShapeDtypeStruct((N_ROWS, 1), jnp.float32),
            jax.ShapeDtypeStruct((N_ROWS, 1), jnp.float32),
        ],
    )(disease_table, gene_table, W, b)
    return u.reshape(N_ROWS), v.reshape(N_ROWS)


def _make_sc_kernel():
    mesh = plsc.VectorSubcoreMesh(core_axis_name="c", subcore_axis_name="s")

    @functools.partial(
        pl.kernel,
        mesh=mesh,
        out_type=jax.ShapeDtypeStruct((BATCH,), jnp.float32),
        scratch_types=[
            pltpu.VMEM((N_CHUNKS, CHUNK), jnp.int32),
            pltpu.VMEM((N_CHUNKS, CHUNK), jnp.int32),
            pltpu.VMEM((N_CHUNKS, CHUNK), jnp.float32),
            pltpu.VMEM((N_CHUNKS, CHUNK), jnp.float32),
            pltpu.VMEM((B_PER_W,), jnp.float32),
            pltpu.SemaphoreType.DMA,
        ],
    )
    def sc_gather(u_hbm, v_hbm, dis_hbm, gene_hbm, out_hbm,
                  idx_d, idx_g, uv, vv, outv, sem):
        wid = lax.axis_index("s") * 2 + lax.axis_index("c")
        base = wid * B_PER_W
        for c in range(N_CHUNKS):
            pltpu.sync_copy(dis_hbm.at[pl.ds(base + c * CHUNK, CHUNK)],
                            idx_d.at[c])
            pltpu.sync_copy(gene_hbm.at[pl.ds(base + c * CHUNK, CHUNK)],
                            idx_g.at[c])
        copies = []
        for c in range(N_CHUNKS):
            copies.append(pltpu.async_copy(u_hbm.at[idx_d.at[c]], uv.at[c], sem))
            copies.append(pltpu.async_copy(v_hbm.at[idx_g.at[c]], vv.at[c], sem))
        for cp in copies:
            cp.wait()
        for c in range(N_CHUNKS):
            for l in range(CHUNK // LANES):
                x = uv[c, pl.ds(l * LANES, LANES)] + vv[c, pl.ds(l * LANES, LANES)]
                outv[pl.ds(c * CHUNK + l * LANES, LANES)] = (
                    1.0 / (1.0 + jnp.exp(-x)))
        pltpu.sync_copy(outv, out_hbm.at[pl.ds(base, B_PER_W)])

    return sc_gather


_sc_gather = _make_sc_kernel()


def kernel(diseases, genes, disease_table, gene_table, W, b):
    u, v = _compute_scores(disease_table, gene_table, W, b)
    return _sc_gather(u, v, diseases, genes)


# speedup vs baseline: 1.0495x; 1.0495x over previous
import functools

import jax
import jax.numpy as jnp
from jax import lax
from jax.experimental import pallas as pl
from jax.experimental.pallas import tpu as pltpu
from jax.experimental.pallas import tpu_sc as plsc

N_ROWS = 100000
N_FACTORS = 64
BATCH = 16384

ROW_BLK = 5000

NUM_WORKERS = 32
B_PER_W = BATCH // NUM_WORKERS
CHUNK = 128
N_CHUNKS = B_PER_W // CHUNK
LANES = 16


def _scores_body(dis_ref, gene_ref, wd_ref, wg_ref, b_ref, u_ref, v_ref):
    x = dis_ref[...]
    y = gene_ref[...]
    u_ref[...] = jnp.dot(x, wd_ref[...],
                         preferred_element_type=jnp.float32) + b_ref[0]
    v_ref[...] = jnp.dot(y, wg_ref[...],
                         preferred_element_type=jnp.float32)


def _compute_scores(disease_table, gene_table, W, b):
    grid = N_ROWS // ROW_BLK
    wd = W[0, :N_FACTORS].reshape(N_FACTORS, 1)
    wg = W[0, N_FACTORS:].reshape(N_FACTORS, 1)
    u, v = pl.pallas_call(
        _scores_body,
        grid=(grid,),
        in_specs=[
            pl.BlockSpec((ROW_BLK, N_FACTORS), lambda i: (i, 0)),
            pl.BlockSpec((ROW_BLK, N_FACTORS), lambda i: (i, 0)),
            pl.BlockSpec((N_FACTORS, 1), lambda i: (0, 0)),
            pl.BlockSpec((N_FACTORS, 1), lambda i: (0, 0)),
            pl.BlockSpec(memory_space=pltpu.SMEM),
        ],
        out_specs=[
            pl.BlockSpec((ROW_BLK, 1), lambda i: (i, 0)),
            pl.BlockSpec((ROW_BLK, 1), lambda i: (i, 0)),
        ],
        out_shape=[
            jax.ShapeDtypeStruct((N_ROWS, 1), jnp.float32),
            jax.ShapeDtypeStruct((N_ROWS, 1), jnp.float32),
        ],
    )(disease_table, gene_table, wd, wg, b)
    return u.reshape(N_ROWS), v.reshape(N_ROWS)


def _make_sc_kernel():
    mesh = plsc.VectorSubcoreMesh(core_axis_name="c", subcore_axis_name="s")

    @functools.partial(
        pl.kernel,
        mesh=mesh,
        out_type=jax.ShapeDtypeStruct((BATCH,), jnp.float32),
        scratch_types=[
            pltpu.VMEM((N_CHUNKS, CHUNK), jnp.int32),
            pltpu.VMEM((N_CHUNKS, CHUNK), jnp.int32),
            pltpu.VMEM((N_CHUNKS, CHUNK), jnp.float32),
            pltpu.VMEM((N_CHUNKS, CHUNK), jnp.float32),
            pltpu.VMEM((B_PER_W,), jnp.float32),
            pltpu.SemaphoreType.DMA,
        ],
    )
    def sc_gather(u_hbm, v_hbm, dis_hbm, gene_hbm, out_hbm,
                  idx_d, idx_g, uv, vv, outv, sem):
        wid = lax.axis_index("s") * 2 + lax.axis_index("c")
        base = wid * B_PER_W
        for c in range(N_CHUNKS):
            pltpu.sync_copy(dis_hbm.at[pl.ds(base + c * CHUNK, CHUNK)],
                            idx_d.at[c])
            pltpu.sync_copy(gene_hbm.at[pl.ds(base + c * CHUNK, CHUNK)],
                            idx_g.at[c])
        copies = []
        for c in range(N_CHUNKS):
            copies.append(pltpu.async_copy(u_hbm.at[idx_d.at[c]], uv.at[c], sem))
            copies.append(pltpu.async_copy(v_hbm.at[idx_g.at[c]], vv.at[c], sem))
        for cp in copies:
            cp.wait()
        for c in range(N_CHUNKS):
            for l in range(CHUNK // LANES):
                x = uv[c, pl.ds(l * LANES, LANES)] + vv[c, pl.ds(l * LANES, LANES)]
                outv[pl.ds(c * CHUNK + l * LANES, LANES)] = (
                    1.0 / (1.0 + jnp.exp(-x)))
        pltpu.sync_copy(outv, out_hbm.at[pl.ds(base, B_PER_W)])

    return sc_gather


_sc_gather = _make_sc_kernel()


def kernel(diseases, genes, disease_table, gene_table, W, b):
    u, v = _compute_scores(disease_table, gene_table, W, b)
    return _sc_gather(u, v, diseases, genes)


# speedup vs baseline: 1.4516x; 1.3832x over previous
import functools

import jax
import jax.numpy as jnp
from jax import lax
from jax.experimental import pallas as pl
from jax.experimental.pallas import tpu as pltpu
from jax.experimental.pallas import tpu_sc as plsc

N_ROWS = 100000
N_FACTORS = 64
BATCH = 16384

NUM_WORKERS = 32
B_PER_W = BATCH // NUM_WORKERS
CHUNK = 128
N_CHUNKS = B_PER_W // CHUNK
LANES = 16
N_GROUPS = B_PER_W // LANES
KSUB = N_FACTORS // LANES


def _make_sc_kernel():
    mesh = plsc.VectorSubcoreMesh(core_axis_name="c", subcore_axis_name="s")

    @functools.partial(
        pl.kernel,
        mesh=mesh,
        compiler_params=pltpu.CompilerParams(needs_layout_passes=False,
                                             use_tc_tiling_on_sc=False),
        out_type=jax.ShapeDtypeStruct((BATCH,), jnp.float32),
        scratch_types=[
            pltpu.VMEM((N_CHUNKS, CHUNK), jnp.int32),
            pltpu.VMEM((N_CHUNKS, CHUNK), jnp.int32),
            pltpu.VMEM((B_PER_W, N_FACTORS), jnp.float32),
            pltpu.VMEM((B_PER_W, N_FACTORS), jnp.float32),
            pltpu.VMEM((2 * N_FACTORS,), jnp.float32),
            pltpu.VMEM((LANES,), jnp.float32),
            pltpu.VMEM((B_PER_W,), jnp.float32),
            pltpu.SemaphoreType.DMA,
        ],
    )
    def sc_body(dis_tab, gene_tab, dis_idx, gene_idx, w_hbm, b_hbm, out_hbm,
                idx_d, idx_g, drows, grows, wv, bv, outv, sem):
        wid = lax.axis_index("s") * 2 + lax.axis_index("c")
        base = wid * B_PER_W
        pltpu.sync_copy(w_hbm, wv)
        pltpu.sync_copy(b_hbm, bv)
        for c in range(N_CHUNKS):
            pltpu.sync_copy(dis_idx.at[pl.ds(base + c * CHUNK, CHUNK)],
                            idx_d.at[c])
            pltpu.sync_copy(gene_idx.at[pl.ds(base + c * CHUNK, CHUNK)],
                            idx_g.at[c])
        copies = []
        for c in range(N_CHUNKS):
            copies.append(pltpu.async_copy(
                dis_tab.at[idx_d.at[c]], drows.at[pl.ds(c * CHUNK, CHUNK)],
                sem))
            copies.append(pltpu.async_copy(
                gene_tab.at[idx_g.at[c]], grows.at[pl.ds(c * CHUNK, CHUNK)],
                sem))
        for cp in copies:
            cp.wait()

        wd = [wv[pl.ds(k * LANES, LANES)] for k in range(KSUB)]
        wg = [wv[pl.ds(N_FACTORS + k * LANES, LANES)] for k in range(KSUB)]
        bvec = bv[...]
        lane = lax.iota(jnp.int32, LANES)

        def group(g, carry):
            acc = bvec
            for r in range(LANES):
                row = g * LANES + r
                p = drows[row, pl.ds(0, LANES)] * wd[0]
                p = p + grows[row, pl.ds(0, LANES)] * wg[0]
                for k in range(1, KSUB):
                    p = p + drows[row, pl.ds(k * LANES, LANES)] * wd[k]
                    p = p + grows[row, pl.ds(k * LANES, LANES)] * wg[k]
                s = jnp.sum(p)
                acc = jnp.where(lane == r, acc + s, acc)
            outv[pl.ds(g * LANES, LANES)] = 1.0 / (1.0 + jnp.exp(-acc))
            return carry

        lax.fori_loop(0, N_GROUPS, group, 0)
        pltpu.sync_copy(outv, out_hbm.at[pl.ds(base, B_PER_W)])

    return sc_body


_sc_kernel = _make_sc_kernel()


def kernel(diseases, genes, disease_table, gene_table, W, b):
    w_flat = W.reshape(2 * N_FACTORS)
    b_vec = jnp.broadcast_to(b, (LANES,))
    return _sc_kernel(disease_table, gene_table, diseases, genes,
                      w_flat, b_vec)
